# Initial kernel scaffold; baseline (speedup 1.0000x reference)
import jax
import jax.numpy as jnp
from jax.experimental import pallas as pl


def kernel(tokens_batch, heads_batch, U, Ubias, V, Vbias):
    raise NotImplementedError("write your pallas kernel here")


# speedup vs baseline: 1.0687x; 1.0687x over previous
import functools

import jax
import jax.numpy as jnp
from jax import lax
from jax.experimental import pallas as pl
from jax.experimental.pallas import tpu as pltpu
from jax.experimental.pallas import tpu_sc as plsc

_VOCAB = 1000000
_DIM = 64
_LANES = 16
_NC = 2
_NS = 16
_NW = _NC * _NS
_GRP = 128
_SUP = 4
_CH = _GRP * _SUP


def _make_sc_kernel(n_groups_total):
    n_groups_w = n_groups_total // _NW
    n_sup = n_groups_w // _SUP
    mesh = plsc.VectorSubcoreMesh(core_axis_name="c", subcore_axis_name="s")

    @functools.partial(
        pl.kernel,
        mesh=mesh,
        out_type=jax.ShapeDtypeStruct((_NW, _LANES), jnp.float32),
        compiler_params=pltpu.CompilerParams(use_tc_tiling_on_sc=False),
        scratch_types=[
            pltpu.VMEM((_SUP, _GRP), jnp.int32),
            pltpu.VMEM((_SUP, _GRP), jnp.int32),
            pltpu.VMEM((_CH, _DIM), jnp.float32),
            pltpu.VMEM((_CH, _DIM), jnp.float32),
            pltpu.VMEM((_CH,), jnp.float32),
            pltpu.VMEM((_CH,), jnp.float32),
            pltpu.VMEM((_LANES,), jnp.float32),
            pltpu.SemaphoreType.DMA,
        ],
    )
    def sc_kernel(tok_hbm, head_hbm, u_hbm, ub_hbm, v_hbm, vb_hbm,
                  out_hbm, idx_t, idx_h, u_rows, v_rows, ub, vb, acc_v, sem):
        wid = lax.axis_index("s") * _NC + lax.axis_index("c")
        g_base = wid * n_groups_w

        def sup_body(sup, acc):
            g0 = g_base + sup * _SUP
            pltpu.sync_copy(tok_hbm.at[pl.ds(g0, _SUP)], idx_t)
            pltpu.sync_copy(head_hbm.at[pl.ds(g0, _SUP)], idx_h)
            copies = []
            for g in range(_SUP):
                dst = pl.ds(g * _GRP, _GRP)
                copies.append(pltpu.async_copy(
                    u_hbm.at[idx_t.at[g]], u_rows.at[dst], sem))
                copies.append(pltpu.async_copy(
                    v_hbm.at[idx_h.at[g]], v_rows.at[dst], sem))
                copies.append(pltpu.async_copy(
                    ub_hbm.at[idx_t.at[g]], ub.at[dst], sem))
                copies.append(pltpu.async_copy(
                    vb_hbm.at[idx_h.at[g]], vb.at[dst], sem))
            for c in copies:
                c.wait()

            def row_body(i, a):
                for s in range(_DIM // _LANES):
                    sl = pl.ds(s * _LANES, _LANES)
                    a = a + u_rows[i, sl] * v_rows[i, sl]
                return a

            acc = lax.fori_loop(0, _CH, row_body, acc)

            def bias_body(j, a):
                sl = pl.ds(j * _LANES, _LANES)
                return a + ub[sl] + vb[sl]

            return lax.fori_loop(0, _CH // _LANES, bias_body, acc)

        acc = lax.fori_loop(0, n_sup, sup_body,
                            jnp.zeros((_LANES,), jnp.float32))
        acc_v[...] = acc
        pltpu.sync_copy(acc_v, out_hbm.at[wid])

    return sc_kernel


def kernel(tokens_batch, heads_batch, U, Ubias, V, Vbias):
    b, l = tokens_batch.shape
    n = b * l
    n_groups_total = n // _GRP
    tok = tokens_batch.reshape(n_groups_total, _GRP).astype(jnp.int32)
    head = heads_batch.reshape(n_groups_total, _GRP).astype(jnp.int32)
    ub_flat = Ubias.reshape(-1)
    vb_flat = Vbias.reshape(-1)
    partials = _make_sc_kernel(n_groups_total)(
        tok, head, U, ub_flat, V, vb_flat)
    return jnp.sum(partials)


# speedup vs baseline: 1.2095x; 1.1318x over previous
import functools

import jax
import jax.numpy as jnp
from jax import lax
from jax.experimental import pallas as pl
from jax.experimental.pallas import tpu as pltpu
from jax.experimental.pallas import tpu_sc as plsc

_VOCAB = 1000000
_DIM = 64
_LANES = 16
_NC = 2
_NS = 16
_NW = _NC * _NS
_GRP = 128
_SUP = 2
_CH = _GRP * _SUP
_NACC = 8
_RU = 2


def _make_sc_kernel(n_groups_total):
    n_groups_w = n_groups_total // _NW
    n_sup = n_groups_w // _SUP
    assert n_sup % 2 == 0
    mesh = plsc.VectorSubcoreMesh(core_axis_name="c", subcore_axis_name="s")

    @functools.partial(
        pl.kernel,
        mesh=mesh,
        out_type=jax.ShapeDtypeStruct((_NW, _LANES), jnp.float32),
        compiler_params=pltpu.CompilerParams(use_tc_tiling_on_sc=False),
        scratch_types=[
            pltpu.VMEM((n_groups_w, _GRP), jnp.int32),
            pltpu.VMEM((n_groups_w, _GRP), jnp.int32),
            pltpu.VMEM((2, _CH, _DIM), jnp.float32),
            pltpu.VMEM((2, _CH, _DIM), jnp.float32),
            pltpu.VMEM((2, _CH), jnp.float32),
            pltpu.VMEM((2, _CH), jnp.float32),
            pltpu.VMEM((_LANES,), jnp.float32),
            pltpu.SemaphoreType.DMA,
            pltpu.SemaphoreType.DMA,
        ],
    )
    def sc_kernel(tok_hbm, head_hbm, u_hbm, ub_hbm, v_hbm, vb_hbm,
                  out_hbm, idx_t, idx_h, u_rows, v_rows, ub, vb, acc_v,
                  sem0, sem1):
        wid = lax.axis_index("s") * _NC + lax.axis_index("c")
        g_base = wid * n_groups_w
        pltpu.sync_copy(tok_hbm.at[pl.ds(g_base, n_groups_w)], idx_t)
        pltpu.sync_copy(head_hbm.at[pl.ds(g_base, n_groups_w)], idx_h)
        sems = (sem0, sem1)

        def issue(sup, b):
            for g in range(_SUP):
                gi = sup * _SUP + g
                dst = pl.ds(g * _GRP, _GRP)
                pltpu.async_copy(u_hbm.at[idx_t.at[gi]],
                                 u_rows.at[b, dst], sems[b])
                pltpu.async_copy(v_hbm.at[idx_h.at[gi]],
                                 v_rows.at[b, dst], sems[b])
                pltpu.async_copy(ub_hbm.at[idx_t.at[gi]],
                                 ub.at[b, dst], sems[b])
                pltpu.async_copy(vb_hbm.at[idx_h.at[gi]],
                                 vb.at[b, dst], sems[b])

        def drain(b):
            for g in range(_SUP):
                dst = pl.ds(g * _GRP, _GRP)
                pltpu.make_async_copy(u_hbm.at[pl.ds(0, _GRP)],
                                      u_rows.at[b, dst], sems[b]).wait()
                pltpu.make_async_copy(v_hbm.at[pl.ds(0, _GRP)],
                                      v_rows.at[b, dst], sems[b]).wait()
                pltpu.make_async_copy(ub_hbm.at[pl.ds(0, _GRP)],
                                      ub.at[b, dst], sems[b]).wait()
                pltpu.make_async_copy(vb_hbm.at[pl.ds(0, _GRP)],
                                      vb.at[b, dst], sems[b]).wait()

        def compute(b, accs):
            def row_body(i, a):
                a = list(a)
                for r in range(_RU):
                    for s in range(_DIM // _LANES):
                        sl = pl.ds(s * _LANES, _LANES)
                        k = r * (_DIM // _LANES) + s
                        a[k] = a[k] + (u_rows[b, i * _RU + r, sl] *
                                       v_rows[b, i * _RU + r, sl])
                return tuple(a)

            accs = lax.fori_loop(0, _CH // _RU, row_body, accs)

            def bias_body(j, a):
                a = list(a)
                a[0] = a[0] + ub[b, pl.ds(j * 2 * _LANES, _LANES)]
                a[1] = a[1] + ub[b, pl.ds((j * 2 + 1) * _LANES, _LANES)]
                a[2] = a[2] + vb[b, pl.ds(j * 2 * _LANES, _LANES)]
                a[3] = a[3] + vb[b, pl.ds((j * 2 + 1) * _LANES, _LANES)]
                return tuple(a)

            return lax.fori_loop(0, _CH // (2 * _LANES), bias_body, accs)

        issue(0, 0)

        def pair_body(p, accs):
            issue(p * 2 + 1, 1)
            drain(0)
            accs = compute(0, accs)
            issue(jnp.minimum(p * 2 + 2, n_sup - 1), 0)
            drain(1)
            return compute(1, accs)

        accs = lax.fori_loop(
            0, n_sup // 2, pair_body,
            tuple(jnp.zeros((_LANES,), jnp.float32) for _ in range(_NACC)))
        drain(0)

        total = accs[0]
        for a in accs[1:]:
            total = total + a
        acc_v[...] = total
        pltpu.sync_copy(acc_v, out_hbm.at[wid])

    return sc_kernel


def kernel(tokens_batch, heads_batch, U, Ubias, V, Vbias):
    b, l = tokens_batch.shape
    n = b * l
    n_groups_total = n // _GRP
    tok = tokens_batch.reshape(n_groups_total, _GRP).astype(jnp.int32)
    head = heads_batch.reshape(n_groups_total, _GRP).astype(jnp.int32)
    ub_flat = Ubias.reshape(-1)
    vb_flat = Vbias.reshape(-1)
    partials = _make_sc_kernel(n_groups_total)(
        tok, head, U, ub_flat, V, vb_flat)
    return jnp.sum(partials)
